# baseline (speedup 1.0000x reference)
import functools

import jax
import jax.numpy as jnp
from jax import lax
from jax.experimental import pallas as pl
from jax.experimental.pallas import tpu as pltpu
from jax.experimental.pallas import tpu_sc as plsc

BATCH = 4096
SEQ = 200
EMBED_DIM = 32
VOCAB = 1000000

GROUP = 1 << 18
NGROUP = 4
CBLK = 4096
TC_GRID = GROUP // CBLK

NUM_CORES = 2
NUM_SUBCORES = 16
NUM_WORKERS = NUM_CORES * NUM_SUBCORES
BPW = BATCH // NUM_WORKERS
PITCH_PAD = 1


def _tp_body(w0, w1, w2, w3, out_ref):
    eye = jnp.eye(EMBED_DIM, dtype=jnp.float32)
    for q, wq in enumerate((w0, w1, w2, w3)):
        t = jax.lax.dot_general(
            wq[...],
            eye,
            dimension_numbers=(((0,), (0,)), ((), ())),
            preferred_element_type=jnp.float32,
        )
        out_ref[:, q * EMBED_DIM:(q + 1) * EMBED_DIM] = t


MAX_CBLK = (VOCAB - 1) // CBLK


def _tc_transpose(wt):
    def spec(q):
        return pl.BlockSpec(
            (EMBED_DIM, CBLK),
            lambda g, q=q: (0, jnp.minimum(q * TC_GRID + g, MAX_CBLK)),
        )

    return pl.pallas_call(
        _tp_body,
        grid=(TC_GRID,),
        in_specs=[spec(0), spec(1), spec(2), spec(3)],
        out_specs=pl.BlockSpec((CBLK, NGROUP * EMBED_DIM), lambda g: (g, 0)),
        out_shape=jax.ShapeDtypeStruct((GROUP, NGROUP * EMBED_DIM), jnp.float32),
    )(wt, wt, wt, wt)


_mesh = plsc.VectorSubcoreMesh(core_axis_name="c", subcore_axis_name="s")


@functools.partial(
    pl.kernel,
    mesh=_mesh,
    out_type=jax.ShapeDtypeStruct((SEQ, EMBED_DIM, BATCH), jnp.float32),
    scratch_types=[
        pltpu.VMEM((SEQ, BPW), jnp.int32),
        pltpu.VMEM((2, BPW, EMBED_DIM), jnp.float32),
        pltpu.VMEM((2, EMBED_DIM, BPW + PITCH_PAD), jnp.float32),
        pltpu.SemaphoreType.DMA,
        pltpu.SemaphoreType.DMA,
    ],
    compiler_params=pltpu.CompilerParams(
        use_tc_tiling_on_sc=False, needs_layout_passes=False
    ),
)
def _sc_gather(table_hbm, idx_hbm, out_hbm, idx_v, rows_v, o2_v, sem_g, sem_o):
    wid = lax.axis_index("s") * NUM_CORES + lax.axis_index("c")
    b0 = wid * BPW
    pltpu.sync_copy(idx_hbm.at[:, pl.ds(b0, BPW)], idx_v)

    def start_gather(s, buf):
        pltpu.async_copy(table_hbm.at[idx_v.at[s]], rows_v.at[buf], sem_g)

    def wait_gather(buf):
        pltpu.make_async_copy(
            table_hbm.at[idx_v.at[0]], rows_v.at[buf], sem_g
        ).wait()

    def start_out(s, buf):
        pltpu.async_copy(
            o2_v.at[buf, :, pl.ds(0, BPW)],
            out_hbm.at[s, :, pl.ds(b0, BPW)],
            sem_o,
        )

    def wait_out(buf):
        pltpu.make_async_copy(
            o2_v.at[buf, :, pl.ds(0, BPW)],
            out_hbm.at[0, :, pl.ds(b0, BPW)],
            sem_o,
        ).wait()

    iota16 = lax.iota(jnp.int32, 16)
    j_lo = iota16
    j_hi = iota16 + 16

    def transpose(buf):
        for r in range(BPW):
            v_lo = rows_v[buf, r, 0:16]
            v_hi = rows_v[buf, r, 16:32]
            r_splat = jnp.full((16,), r, jnp.int32)
            plsc.store_scatter(o2_v.at[buf], [j_lo, r_splat], v_lo)
            plsc.store_scatter(o2_v.at[buf], [j_hi, r_splat], v_hi)

    def process(s, buf):
        wait_gather(buf)

        @pl.when(s + 1 < SEQ)
        def _():
            start_gather(s + 1, 1 - buf)

        @pl.when(s >= 2)
        def _():
            wait_out(buf)

        transpose(buf)
        start_out(s, buf)

    start_gather(0, 0)

    def body(t, carry):
        process(2 * t, 0)
        process(2 * t + 1, 1)
        return carry

    lax.fori_loop(0, SEQ // 2, body, 0)
    wait_out(0)
    wait_out(1)


def kernel(x, weight):
    xT = x.T.astype(jnp.int32)
    idxk = ((xT & (GROUP - 1)) << 2) | (xT >> 18)
    table2 = _tc_transpose(weight.T).reshape(GROUP * NGROUP, EMBED_DIM)
    o2 = _sc_gather(table2, idxk)
    return o2.transpose(2, 0, 1)


# speedup vs baseline: 1.0088x; 1.0088x over previous
import functools

import jax
import jax.numpy as jnp
from jax import lax
from jax.experimental import pallas as pl
from jax.experimental.pallas import tpu as pltpu
from jax.experimental.pallas import tpu_sc as plsc

BATCH = 4096
SEQ = 200
EMBED_DIM = 32
VOCAB = 1000000

GROUP = 1 << 18
NGROUP = 4
CBLK = 8192
TC_GRID = GROUP // CBLK

NUM_CORES = 2
NUM_SUBCORES = 16
NUM_WORKERS = NUM_CORES * NUM_SUBCORES
BPW = BATCH // NUM_WORKERS
PITCH_PAD = 1


def _tp_body(w0, w1, w2, w3, out_ref):
    for q, wq in enumerate((w0, w1, w2, w3)):
        out_ref[:, q * EMBED_DIM:(q + 1) * EMBED_DIM] = jnp.transpose(wq[...])


MAX_CBLK = (VOCAB - 1) // CBLK


def _tc_transpose(wt):
    def spec(q):
        return pl.BlockSpec(
            (EMBED_DIM, CBLK),
            lambda g, q=q: (0, jnp.minimum(q * TC_GRID + g, MAX_CBLK)),
        )

    return pl.pallas_call(
        _tp_body,
        grid=(TC_GRID,),
        in_specs=[spec(0), spec(1), spec(2), spec(3)],
        out_specs=pl.BlockSpec((CBLK, NGROUP * EMBED_DIM), lambda g: (g, 0)),
        out_shape=jax.ShapeDtypeStruct((GROUP, NGROUP * EMBED_DIM), jnp.float32),
    )(wt, wt, wt, wt)


_mesh = plsc.VectorSubcoreMesh(core_axis_name="c", subcore_axis_name="s")


@functools.partial(
    pl.kernel,
    mesh=_mesh,
    out_type=jax.ShapeDtypeStruct((SEQ, EMBED_DIM, BATCH), jnp.float32),
    scratch_types=[
        pltpu.VMEM((SEQ, BPW), jnp.int32),
        pltpu.VMEM((2, BPW, EMBED_DIM), jnp.float32),
        pltpu.VMEM((2, EMBED_DIM, BPW + PITCH_PAD), jnp.float32),
        pltpu.SemaphoreType.DMA,
        pltpu.SemaphoreType.DMA,
    ],
    compiler_params=pltpu.CompilerParams(
        use_tc_tiling_on_sc=False, needs_layout_passes=False
    ),
)
def _sc_gather(table_hbm, idx_hbm, out_hbm, idx_v, rows_v, o2_v, sem_g, sem_o):
    wid = lax.axis_index("s") * NUM_CORES + lax.axis_index("c")
    b0 = wid * BPW
    pltpu.sync_copy(idx_hbm.at[:, pl.ds(b0, BPW)], idx_v)

    def start_gather(s, buf):
        pltpu.async_copy(table_hbm.at[idx_v.at[s]], rows_v.at[buf], sem_g)

    def wait_gather(buf):
        pltpu.make_async_copy(
            table_hbm.at[idx_v.at[0]], rows_v.at[buf], sem_g
        ).wait()

    def start_out(s, buf):
        pltpu.async_copy(
            o2_v.at[buf, :, pl.ds(0, BPW)],
            out_hbm.at[s, :, pl.ds(b0, BPW)],
            sem_o,
        )

    def wait_out(buf):
        pltpu.make_async_copy(
            o2_v.at[buf, :, pl.ds(0, BPW)],
            out_hbm.at[0, :, pl.ds(b0, BPW)],
            sem_o,
        ).wait()

    iota16 = lax.iota(jnp.int32, 16)
    j_lo = iota16
    j_hi = iota16 + 16

    def transpose(buf):
        for r in range(BPW):
            v_lo = rows_v[buf, r, 0:16]
            v_hi = rows_v[buf, r, 16:32]
            r_splat = jnp.full((16,), r, jnp.int32)
            plsc.store_scatter(o2_v.at[buf], [j_lo, r_splat], v_lo)
            plsc.store_scatter(o2_v.at[buf], [j_hi, r_splat], v_hi)

    def process(s, buf):
        wait_gather(buf)

        @pl.when(s + 1 < SEQ)
        def _():
            start_gather(s + 1, 1 - buf)

        @pl.when(s >= 2)
        def _():
            wait_out(buf)

        transpose(buf)
        start_out(s, buf)

    start_gather(0, 0)

    def body(t, carry):
        process(2 * t, 0)
        process(2 * t + 1, 1)
        return carry

    lax.fori_loop(0, SEQ // 2, body, 0)
    wait_out(0)
    wait_out(1)


def kernel(x, weight):
    xT = x.T.astype(jnp.int32)
    idxk = ((xT & (GROUP - 1)) << 2) | (xT >> 18)
    table2 = _tc_transpose(weight.T).reshape(GROUP * NGROUP, EMBED_DIM)
    o2 = _sc_gather(table2, idxk)
    return o2.transpose(2, 0, 1)
